# Initial kernel scaffold; baseline (speedup 1.0000x reference)
import jax
import jax.numpy as jnp
from jax.experimental import pallas as pl


def kernel(ids, table):
    raise NotImplementedError("write your pallas kernel here")


# speedup vs baseline: 1.5755x; 1.5755x over previous
import functools

import jax
import jax.numpy as jnp
from jax import lax
from jax.experimental import pallas as pl
from jax.experimental.pallas import tpu as pltpu
from jax.experimental.pallas import tpu_sc as plsc

EMBEDDING_DIM = 32
CHUNK = 128
GROUP = 8
ROWS_G = CHUNK * GROUP

_NUM_CORES = 2
_NUM_SUBCORES = 16
_NUM_WORKERS = _NUM_CORES * _NUM_SUBCORES


@functools.lru_cache(maxsize=None)
def _make_gather(total_rows: int, dim: int):
    assert total_rows % (ROWS_G * _NUM_WORKERS) == 0
    chunks_per_worker = total_rows // (CHUNK * _NUM_WORKERS)
    groups_per_worker = chunks_per_worker // GROUP
    assert groups_per_worker >= 3 and groups_per_worker % 2 == 1
    mesh = plsc.VectorSubcoreMesh(core_axis_name="c", subcore_axis_name="s")

    @functools.partial(
        pl.kernel,
        mesh=mesh,
        out_type=jax.ShapeDtypeStruct((total_rows, dim), jnp.float32),
        scratch_types=[
            pltpu.VMEM((chunks_per_worker, CHUNK), jnp.int32),
            pltpu.VMEM((ROWS_G, dim), jnp.float32),
            pltpu.VMEM((ROWS_G, dim), jnp.float32),
            pltpu.SemaphoreType.DMA,
            pltpu.SemaphoreType.DMA,
            pltpu.SemaphoreType.DMA,
            pltpu.SemaphoreType.DMA,
        ],
        compiler_params=pltpu.CompilerParams(use_tc_tiling_on_sc=False),
    )
    def gather_kernel(ids_hbm, table_hbm, out_hbm, idx_v, buf_a, buf_b,
                      ga, gb, oa, ob):
        wid = lax.axis_index("s") * _NUM_CORES + lax.axis_index("c")
        base_chunk = wid * chunks_per_worker
        pltpu.sync_copy(ids_hbm.at[pl.ds(base_chunk, chunks_per_worker)], idx_v)

        def fire_group(g, buf, gsem):
            for b in range(GROUP):
                pltpu.async_copy(
                    table_hbm.at[idx_v.at[g * GROUP + b]],
                    buf.at[pl.ds(b * CHUNK, CHUNK)],
                    gsem,
                )

        def drain_gathers(buf, gsem):
            pltpu.make_async_copy(
                table_hbm.at[pl.ds(0, ROWS_G)], buf, gsem).wait()

        def fire_out(g, buf, osem):
            pltpu.async_copy(
                buf,
                out_hbm.at[pl.ds((base_chunk + g * GROUP) * CHUNK, ROWS_G)],
                osem,
            )

        def drain_out(buf, osem):
            pltpu.make_async_copy(
                buf, out_hbm.at[pl.ds(0, ROWS_G)], osem).wait()

        last = groups_per_worker - 1

        fire_group(0, buf_a, ga)
        fire_group(1, buf_b, gb)
        drain_gathers(buf_a, ga)
        fire_out(0, buf_a, oa)

        def pair(k, carry):
            g1 = 2 * k + 1
            drain_out(buf_a, oa)
            fire_group(g1 + 1, buf_a, ga)
            drain_gathers(buf_b, gb)
            fire_out(g1, buf_b, ob)
            g2 = 2 * k + 2
            drain_out(buf_b, ob)
            fire_group(g2 + 1, buf_b, gb)
            drain_gathers(buf_a, ga)
            fire_out(g2, buf_a, oa)
            return carry

        lax.fori_loop(0, (groups_per_worker - 3) // 2, pair, 0)

        drain_out(buf_a, oa)
        fire_group(last, buf_a, ga)
        drain_gathers(buf_b, gb)
        fire_out(last - 1, buf_b, ob)

        drain_out(buf_b, ob)
        drain_gathers(buf_a, ga)
        fire_out(last, buf_a, oa)
        drain_out(buf_a, oa)

    return gather_kernel


def kernel(ids, table):
    batch, n_fields = ids.shape
    total = batch * n_fields
    ids2d = ids.reshape(total // CHUNK, CHUNK).astype(jnp.int32)
    out = _make_gather(total, table.shape[1])(ids2d, table)
    return out.reshape(batch, n_fields, table.shape[1])
